# baseline (speedup 1.0000x reference)
import jax
import jax.numpy as jnp
from jax.experimental import pallas as pl
from jax.experimental.pallas import tpu as pltpu

B, L, D = 16, 4096, 1024


def _body(mask_ref, feats_ref, out_ref):
    x = feats_ref[0]
    m = mask_ref[0]
    mb = m > 0.5
    neg = jnp.float32(-1e30)
    maxv = jnp.max(jnp.where(mb, x, neg), axis=0)
    sumv = jnp.sum(x * m, axis=0)
    cnt = jnp.sum(m)
    out_ref[0, 0, :D] = maxv
    out_ref[0, 0, D:] = sumv / cnt


def kernel(feats, mask):
    maskf = mask.astype(jnp.float32).reshape(B, L, 1)
    out = pl.pallas_call(
        _body,
        grid=(B,),
        in_specs=[
            pl.BlockSpec((1, L, 1), lambda b: (b, 0, 0)),
            pl.BlockSpec((1, L, D), lambda b: (b, 0, 0)),
        ],
        out_specs=pl.BlockSpec((1, 1, 2 * D), lambda b: (b, 0, 0)),
        out_shape=jax.ShapeDtypeStruct((B, 1, 2 * D), jnp.float32),
    )(maskf, feats)
    return out.reshape(B, 2 * D)


# speedup vs baseline: 1.1071x; 1.1071x over previous
import jax
import jax.numpy as jnp
from jax.experimental import pallas as pl
from jax.experimental.pallas import tpu as pltpu

B, L, D = 16, 4096, 1024


def _body(mask_ref, feats_ref, out_ref):
    x = feats_ref[0]
    m = mask_ref[0].astype(jnp.float32)
    mb = m > 0.5
    neg = jnp.float32(-1e30)
    maxv = jnp.max(jnp.where(mb, x, neg), axis=0)
    sumv = jnp.sum(jnp.where(mb, x, 0.0), axis=0)
    cnt = jnp.sum(m)
    out_ref[0, 0, :D] = maxv
    out_ref[0, 0, D:] = sumv / cnt


def kernel(feats, mask):
    maskf = mask.astype(jnp.int8).reshape(B, L, 1)
    out = pl.pallas_call(
        _body,
        grid=(B,),
        in_specs=[
            pl.BlockSpec((1, L, 1), lambda b: (b, 0, 0)),
            pl.BlockSpec((1, L, D), lambda b: (b, 0, 0)),
        ],
        out_specs=pl.BlockSpec((1, 1, 2 * D), lambda b: (b, 0, 0)),
        out_shape=jax.ShapeDtypeStruct((B, 1, 2 * D), jnp.float32),
    )(maskf, feats)
    return out.reshape(B, 2 * D)
